# baseline (speedup 1.0000x reference)
import functools

import jax
import jax.numpy as jnp
from jax import lax
from jax.experimental import pallas as pl
from jax.experimental.pallas import tpu as pltpu
from jax.experimental.pallas import tpu_sc as plsc

L = 16
TM = 512
NT = 16
NP = NT * TM


def _route_body(E, logits_ref, pos0_ref, pos1_ref, s0x_ref, s1x_ref, meta_ref):
    lg = logits_ref[...].astype(jnp.float32)
    T = lg.shape[0]
    iota = lax.broadcasted_iota(jnp.int32, (T, E), 1)
    v1 = jnp.max(lg, axis=1, keepdims=True)
    e1 = jnp.min(jnp.where(lg == v1, iota, E), axis=1, keepdims=True)
    lm = jnp.where(iota == e1, -3e38, lg)
    v2 = jnp.max(lm, axis=1, keepdims=True)
    e2 = jnp.min(jnp.where(lm == v2, iota, E), axis=1, keepdims=True)
    q = jnp.exp(v2 - v1)
    s1 = 1.0 / (1.0 + q)
    s2 = 1.0 - s1

    oh1 = (iota == e1).astype(jnp.bfloat16)
    oh2 = (iota == e2).astype(jnp.bfloat16)
    oh = oh1 + oh2
    r_i = lax.broadcasted_iota(jnp.int32, (T, T), 0)
    c_i = lax.broadcasted_iota(jnp.int32, (T, T), 1)
    tri = (c_i < r_i).astype(jnp.bfloat16)
    ranks = lax.dot_general(tri, oh, (((1,), (0,)), ((), ())),
                            preferred_element_type=jnp.float32)
    cnt = jnp.sum(oh.astype(jnp.float32), axis=0, keepdims=True)
    ptiles = jnp.ceil(cnt / TM)
    ce = lax.broadcasted_iota(jnp.int32, (E, E), 0)
    re = lax.broadcasted_iota(jnp.int32, (E, E), 1)
    trie = (ce < re).astype(jnp.float32)
    tstart = lax.dot_general(ptiles, trie, (((1,), (0,)), ((), ())),
                             preferred_element_type=jnp.float32)
    po = tstart * TM
    slot = po + ranks
    pos0 = jnp.sum(jnp.where(iota == e1, slot, 0.0), axis=1, keepdims=True)
    pos1 = jnp.sum(jnp.where(iota == e2, slot, 0.0), axis=1, keepdims=True)
    pos0_ref[...] = jnp.broadcast_to(pos0.astype(jnp.int32), (T, 8))
    pos1_ref[...] = jnp.broadcast_to(pos1.astype(jnp.int32), (T, 8))
    s0x_ref[...] = jnp.broadcast_to(s1, (T, 128)).astype(jnp.float32)
    s1x_ref[...] = jnp.broadcast_to(s2, (T, 128)).astype(jnp.float32)

    tcum = tstart + ptiles
    ident = (ce == re).astype(jnp.float32)
    tcol = lax.dot_general(ident, tcum, (((1,), (1,)), ((), ())),
                           preferred_element_type=jnp.float32)
    ntiles = jnp.sum(ptiles).astype(jnp.int32)
    jrow = lax.broadcasted_iota(jnp.int32, (E, 32), 1)
    texp = jnp.sum((tcol.astype(jnp.int32) <= jrow).astype(jnp.int32),
                   axis=0, keepdims=True)
    j32 = lax.broadcasted_iota(jnp.int32, (1, 32), 1)
    lastexp = jnp.sum(jnp.where(j32 == ntiles - 1, texp, 0))
    texp = jnp.minimum(texp, lastexp)
    mrow = jnp.where(j32 == NT, ntiles, jnp.where(j32 < NT, texp, 0))
    cnt_t = jnp.zeros((1, 32), jnp.float32)
    tst_t = jnp.zeros((1, 32), jnp.float32)
    iota_e = lax.broadcasted_iota(jnp.int32, (1, E), 1)
    for e in range(E):
        ce_s = jnp.sum(jnp.where(iota_e == e, cnt, 0.0))
        ts_s = jnp.sum(jnp.where(iota_e == e, tstart, 0.0))
        sel = texp == e
        cnt_t = cnt_t + jnp.where(sel, ce_s, 0.0)
        tst_t = tst_t + jnp.where(sel, ts_s, 0.0)
    rows = jnp.clip(cnt_t - (j32.astype(jnp.float32) - tst_t) * TM, 0.0, TM)
    rows = jnp.where(j32 < ntiles, rows, 0.0).astype(jnp.int32)
    si = lax.broadcasted_iota(jnp.int32, (8, 32), 0)
    meta_ref[...] = jnp.where(si == 0, jnp.broadcast_to(mrow, (8, 32)),
                              jnp.broadcast_to(rows, (8, 32)))


def _scatter_body(T, H, pos0_hbm, pos1_hbm, s0_hbm, s1_hbm, x_hbm,
                  xs_hbm, ss_hbm, posb0, posb1, sbuf0, sbuf1, xbuf,
                  sem, sem2, sem3):
    cid = lax.axis_index("c")
    sid = lax.axis_index("s")
    wid = sid * 2 + cid
    tpw = T // 32
    t0 = wid * tpw
    cx = pltpu.async_copy(x_hbm.at[pl.ds(t0, tpw)], xbuf, sem)
    c0 = pltpu.async_copy(s0_hbm.at[pl.ds(t0, tpw)], sbuf0, sem2)
    c1 = pltpu.async_copy(s1_hbm.at[pl.ds(t0, tpw)], sbuf1, sem3)
    pltpu.sync_copy(pos0_hbm.at[pl.ds(t0, tpw)], posb0.at[0])
    pltpu.sync_copy(pos1_hbm.at[pl.ds(t0, tpw)], posb1.at[0])
    c0.wait()
    c1.wait()
    cx.wait()
    w0 = pltpu.async_copy(sbuf0, ss_hbm.at[posb0.at[0]], sem2)
    w1 = pltpu.async_copy(sbuf1, ss_hbm.at[posb1.at[0]], sem3)
    ca = pltpu.async_copy(xbuf, xs_hbm.at[posb0.at[0]], sem)
    cb = pltpu.async_copy(xbuf, xs_hbm.at[posb1.at[0]], sem)
    w0.wait()
    w1.wait()
    ca.wait()
    cb.wait()


def _gemm_body(I, meta_ref, xs_ref, ss_ref, w31_ref, w2_ref, ys_ref):
    t = pl.program_id(0)
    rows = meta_ref[32 + t]
    QS = TM // 4

    @pl.when(rows > 0)
    def _():
        w31b = w31_ref[0].astype(jnp.bfloat16)
        w2b = w2_ref[0].astype(jnp.bfloat16)
        dn = (((1,), (1,)), ((), ()))
        for q in range(4):
            @pl.when(rows > q * QS)
            def _(q=q, w31b=w31b, w2b=w2b, dn=dn):
                sl = pl.ds(q * QS, QS)
                xb = xs_ref[sl, :].astype(jnp.bfloat16)
                h = lax.dot_general(xb, w31b, dn,
                                    preferred_element_type=jnp.float32)
                x3 = h[:, :I]
                x1 = h[:, I:]
                a = (x1 * jax.nn.sigmoid(x1)) * x3
                y = lax.dot_general(a.astype(jnp.bfloat16), w2b, dn,
                                    preferred_element_type=jnp.float32)
                ys_ref[sl, :] = y * ss_ref[sl, 0:1]


def _combine_body(T, H, ys_hbm, pos0_hbm, pos1_hbm, out_hbm,
                  p0v, p1v, bufA, bufB, semA, semB):
    cid = lax.axis_index("c")
    sid = lax.axis_index("s")
    wid = sid * 2 + cid
    tpw = T // 32
    base = wid * tpw
    pltpu.sync_copy(pos0_hbm.at[pl.ds(base, tpw)], p0v.at[0])
    pltpu.sync_copy(pos1_hbm.at[pl.ds(base, tpw)], p1v.at[0])
    ca = pltpu.async_copy(ys_hbm.at[p0v.at[0]], bufA, semA)
    cb = pltpu.async_copy(ys_hbm.at[p1v.at[0]], bufB, semB)
    ca.wait()
    cb.wait()
    for r in range(64):
        def inner(k4, _, r=r):
            for u in range(4):
                sl = pl.ds(k4 * 4 * L + u * L, L)
                bufA[r, sl] = bufA[r, sl] + bufB[r, sl]
            return 0

        lax.fori_loop(0, H // (4 * L), inner, 0)
    pltpu.sync_copy(bufA, out_hbm.at[pl.ds(base, tpw)])


def kernel(x, router_logits, w3_w1_weight, w2_weight):
    T, H = x.shape
    E = w3_w1_weight.shape[0]
    I = w3_w1_weight.shape[1] // 2

    pos0f, pos1f, s0x, s1x, meta8 = pl.pallas_call(
        functools.partial(_route_body, E),
        out_shape=[
            jax.ShapeDtypeStruct((T, 8), jnp.int32),
            jax.ShapeDtypeStruct((T, 8), jnp.int32),
            jax.ShapeDtypeStruct((T, 128), jnp.float32),
            jax.ShapeDtypeStruct((T, 128), jnp.float32),
            jax.ShapeDtypeStruct((8, 32), jnp.int32),
        ],
    )(router_logits)
    pos0 = pos0f[:, 0]
    pos1 = pos1f[:, 0]
    s0c = s0x
    s1c = s1x
    meta = meta8[:2].reshape(64)

    mesh = plsc.VectorSubcoreMesh(core_axis_name="c", subcore_axis_name="s")

    scatter = functools.partial(
        pl.kernel,
        mesh=mesh,
        out_type=[
            jax.ShapeDtypeStruct((NP, H), jnp.float32),
            jax.ShapeDtypeStruct((NP, 128), jnp.float32),
        ],
        scratch_types=[
            pltpu.VMEM((1, T // 32), jnp.int32),
            pltpu.VMEM((1, T // 32), jnp.int32),
            pltpu.VMEM((T // 32, 128), jnp.float32),
            pltpu.VMEM((T // 32, 128), jnp.float32),
            pltpu.VMEM((T // 32, H), jnp.float32),
            pltpu.SemaphoreType.DMA,
            pltpu.SemaphoreType.DMA,
            pltpu.SemaphoreType.DMA,
        ],
    )(functools.partial(_scatter_body, T, H))
    xs, ss = scatter(pos0, pos1, s0c, s1c, x)

    grid_spec = pltpu.PrefetchScalarGridSpec(
        num_scalar_prefetch=1,
        grid=(NT,),
        in_specs=[
            pl.BlockSpec((TM, H), lambda t, m: (t, 0)),
            pl.BlockSpec((TM, 128), lambda t, m: (t, 0)),
            pl.BlockSpec((1, 2 * I, H), lambda t, m: (m[t], 0, 0)),
            pl.BlockSpec((1, H, I), lambda t, m: (m[t], 0, 0)),
        ],
        out_specs=pl.BlockSpec((TM, H), lambda t, m: (t, 0)),
    )
    ys = pl.pallas_call(
        functools.partial(_gemm_body, I),
        grid_spec=grid_spec,
        out_shape=jax.ShapeDtypeStruct((NP, H), jnp.float32),
        compiler_params=pltpu.CompilerParams(
            dimension_semantics=("arbitrary",),
        ),
    )(meta, xs, ss, w3_w1_weight, w2_weight)

    combine = functools.partial(
        pl.kernel,
        mesh=mesh,
        out_type=jax.ShapeDtypeStruct((T, H), jnp.float32),
        scratch_types=[
            pltpu.VMEM((1, T // 32), jnp.int32),
            pltpu.VMEM((1, T // 32), jnp.int32),
            pltpu.VMEM((T // 32, H), jnp.float32),
            pltpu.VMEM((T // 32, H), jnp.float32),
            pltpu.SemaphoreType.DMA,
            pltpu.SemaphoreType.DMA,
        ],
    )(functools.partial(_combine_body, T, H))
    out = combine(ys, pos0, pos1)
    return out


# speedup vs baseline: 1.3489x; 1.3489x over previous
import functools

import jax
import jax.numpy as jnp
from jax import lax
from jax.experimental import pallas as pl
from jax.experimental.pallas import tpu as pltpu
from jax.experimental.pallas import tpu_sc as plsc

L = 16
TM = 512
NT = 16
NP = NT * TM


def _route_body(E, logits_ref, pos0_ref, pos1_ref, s0x_ref, s1x_ref, meta_ref):
    lg = logits_ref[...].astype(jnp.float32)
    T = lg.shape[0]
    iota = lax.broadcasted_iota(jnp.int32, (T, E), 1)
    v1 = jnp.max(lg, axis=1, keepdims=True)
    e1 = jnp.min(jnp.where(lg == v1, iota, E), axis=1, keepdims=True)
    lm = jnp.where(iota == e1, -3e38, lg)
    v2 = jnp.max(lm, axis=1, keepdims=True)
    e2 = jnp.min(jnp.where(lm == v2, iota, E), axis=1, keepdims=True)
    q = jnp.exp(v2 - v1)
    s1 = 1.0 / (1.0 + q)
    s2 = 1.0 - s1

    oh1 = (iota == e1).astype(jnp.bfloat16)
    oh2 = (iota == e2).astype(jnp.bfloat16)
    oh = oh1 + oh2
    r_i = lax.broadcasted_iota(jnp.int32, (T, T), 0)
    c_i = lax.broadcasted_iota(jnp.int32, (T, T), 1)
    tri = (c_i < r_i).astype(jnp.bfloat16)
    ranks = lax.dot_general(tri, oh, (((1,), (0,)), ((), ())),
                            preferred_element_type=jnp.float32)
    cnt = jnp.sum(oh.astype(jnp.float32), axis=0, keepdims=True)
    ptiles = jnp.ceil(cnt / TM)
    ce = lax.broadcasted_iota(jnp.int32, (E, E), 0)
    re = lax.broadcasted_iota(jnp.int32, (E, E), 1)
    trie = (ce < re).astype(jnp.float32)
    tstart = lax.dot_general(ptiles, trie, (((1,), (0,)), ((), ())),
                             preferred_element_type=jnp.float32)
    po = tstart * TM
    slot = po + ranks
    pos0 = jnp.sum(jnp.where(iota == e1, slot, 0.0), axis=1, keepdims=True)
    pos1 = jnp.sum(jnp.where(iota == e2, slot, 0.0), axis=1, keepdims=True)
    pos0_ref[...] = jnp.broadcast_to(pos0.astype(jnp.int32), (T, 8))
    pos1_ref[...] = jnp.broadcast_to(pos1.astype(jnp.int32), (T, 8))
    s0x_ref[...] = jnp.broadcast_to(s1, (T, 128)).astype(jnp.float32)
    s1x_ref[...] = jnp.broadcast_to(s2, (T, 128)).astype(jnp.float32)

    tcum = tstart + ptiles
    ident = (ce == re).astype(jnp.float32)
    tcol = lax.dot_general(ident, tcum, (((1,), (1,)), ((), ())),
                           preferred_element_type=jnp.float32)
    ntiles = jnp.sum(ptiles).astype(jnp.int32)
    jrow = lax.broadcasted_iota(jnp.int32, (E, 32), 1)
    texp = jnp.sum((tcol.astype(jnp.int32) <= jrow).astype(jnp.int32),
                   axis=0, keepdims=True)
    j32 = lax.broadcasted_iota(jnp.int32, (1, 32), 1)
    lastexp = jnp.sum(jnp.where(j32 == ntiles - 1, texp, 0))
    texp = jnp.minimum(texp, lastexp)
    mrow = jnp.where(j32 == NT, ntiles, jnp.where(j32 < NT, texp, 0))
    cnt_t = jnp.zeros((1, 32), jnp.float32)
    tst_t = jnp.zeros((1, 32), jnp.float32)
    iota_e = lax.broadcasted_iota(jnp.int32, (1, E), 1)
    for e in range(E):
        ce_s = jnp.sum(jnp.where(iota_e == e, cnt, 0.0))
        ts_s = jnp.sum(jnp.where(iota_e == e, tstart, 0.0))
        sel = texp == e
        cnt_t = cnt_t + jnp.where(sel, ce_s, 0.0)
        tst_t = tst_t + jnp.where(sel, ts_s, 0.0)
    rows = jnp.clip(cnt_t - (j32.astype(jnp.float32) - tst_t) * TM, 0.0, TM)
    rows = jnp.where(j32 < ntiles, rows, 0.0).astype(jnp.int32)
    si = lax.broadcasted_iota(jnp.int32, (8, 32), 0)
    meta_ref[...] = jnp.where(si == 0, jnp.broadcast_to(mrow, (8, 32)),
                              jnp.broadcast_to(rows, (8, 32)))


def _scatter_body(T, H, pos0_hbm, pos1_hbm, s0_hbm, s1_hbm, x_hbm,
                  xs_hbm, ss_hbm, posb0, posb1, sbuf0, sbuf1, xbuf,
                  sem, sem2, sem3):
    cid = lax.axis_index("c")
    sid = lax.axis_index("s")
    wid = sid * 2 + cid
    tpw = T // 32
    t0 = wid * tpw
    cx = pltpu.async_copy(x_hbm.at[pl.ds(t0, tpw)], xbuf, sem)
    c0 = pltpu.async_copy(s0_hbm.at[pl.ds(t0, tpw)], sbuf0, sem2)
    c1 = pltpu.async_copy(s1_hbm.at[pl.ds(t0, tpw)], sbuf1, sem3)
    pltpu.sync_copy(pos0_hbm.at[pl.ds(t0, tpw)], posb0.at[0])
    pltpu.sync_copy(pos1_hbm.at[pl.ds(t0, tpw)], posb1.at[0])
    c0.wait()
    c1.wait()
    cx.wait()
    w0 = pltpu.async_copy(sbuf0, ss_hbm.at[posb0.at[0]], sem2)
    w1 = pltpu.async_copy(sbuf1, ss_hbm.at[posb1.at[0]], sem3)
    ca = pltpu.async_copy(xbuf, xs_hbm.at[posb0.at[0]], sem)
    cb = pltpu.async_copy(xbuf, xs_hbm.at[posb1.at[0]], sem)
    w0.wait()
    w1.wait()
    ca.wait()
    cb.wait()


def _gemm_body(I, meta_ref, xs_ref, ss_ref, w31_ref, w2_ref, ys_ref):
    t = pl.program_id(0)
    rows = meta_ref[32 + t]
    HS = TM // 2

    @pl.when(rows > 0)
    def _():
        w31b = w31_ref[0].astype(jnp.bfloat16)
        w2b = w2_ref[0].astype(jnp.bfloat16)
        dn = (((1,), (1,)), ((), ()))
        for q in range(2):
            @pl.when(rows > q * HS)
            def _(q=q, w31b=w31b, w2b=w2b, dn=dn):
                sl = pl.ds(q * HS, HS)
                xb = xs_ref[sl, :].astype(jnp.bfloat16)
                h = lax.dot_general(xb, w31b, dn,
                                    preferred_element_type=jnp.float32)
                x3 = h[:, :I]
                x1 = h[:, I:]
                a = (x1 * jax.nn.sigmoid(x1)) * x3
                y = lax.dot_general(a.astype(jnp.bfloat16), w2b, dn,
                                    preferred_element_type=jnp.float32)
                ys_ref[sl, :] = y * ss_ref[sl, 0:1]


def _combine_body(T, H, ys_hbm, pos0_hbm, pos1_hbm, out_hbm,
                  p0v, p1v, bufA, bufB, semA, semB):
    cid = lax.axis_index("c")
    sid = lax.axis_index("s")
    wid = sid * 2 + cid
    tpw = T // 32
    base = wid * tpw
    pltpu.sync_copy(pos0_hbm.at[pl.ds(base, tpw)], p0v.at[0])
    pltpu.sync_copy(pos1_hbm.at[pl.ds(base, tpw)], p1v.at[0])
    ca = pltpu.async_copy(ys_hbm.at[p0v.at[0]], bufA, semA)
    cb = pltpu.async_copy(ys_hbm.at[p1v.at[0]], bufB, semB)
    ca.wait()
    cb.wait()
    for r in range(64):
        def inner(k4, _, r=r):
            for u in range(4):
                sl = pl.ds(k4 * 4 * L + u * L, L)
                bufA[r, sl] = bufA[r, sl] + bufB[r, sl]
            return 0

        lax.fori_loop(0, H // (4 * L), inner, 0)
    pltpu.sync_copy(bufA, out_hbm.at[pl.ds(base, tpw)])


def kernel(x, router_logits, w3_w1_weight, w2_weight):
    T, H = x.shape
    E = w3_w1_weight.shape[0]
    I = w3_w1_weight.shape[1] // 2

    pos0f, pos1f, s0x, s1x, meta8 = pl.pallas_call(
        functools.partial(_route_body, E),
        out_shape=[
            jax.ShapeDtypeStruct((T, 8), jnp.int32),
            jax.ShapeDtypeStruct((T, 8), jnp.int32),
            jax.ShapeDtypeStruct((T, 128), jnp.float32),
            jax.ShapeDtypeStruct((T, 128), jnp.float32),
            jax.ShapeDtypeStruct((8, 32), jnp.int32),
        ],
    )(router_logits)
    pos0 = pos0f[:, 0]
    pos1 = pos1f[:, 0]
    s0c = s0x
    s1c = s1x
    meta = meta8[:2].reshape(64)

    mesh = plsc.VectorSubcoreMesh(core_axis_name="c", subcore_axis_name="s")

    scatter = functools.partial(
        pl.kernel,
        mesh=mesh,
        out_type=[
            jax.ShapeDtypeStruct((NP, H), jnp.float32),
            jax.ShapeDtypeStruct((NP, 128), jnp.float32),
        ],
        scratch_types=[
            pltpu.VMEM((1, T // 32), jnp.int32),
            pltpu.VMEM((1, T // 32), jnp.int32),
            pltpu.VMEM((T // 32, 128), jnp.float32),
            pltpu.VMEM((T // 32, 128), jnp.float32),
            pltpu.VMEM((T // 32, H), jnp.float32),
            pltpu.SemaphoreType.DMA,
            pltpu.SemaphoreType.DMA,
            pltpu.SemaphoreType.DMA,
        ],
    )(functools.partial(_scatter_body, T, H))
    xs, ss = scatter(pos0, pos1, s0c, s1c, x)

    grid_spec = pltpu.PrefetchScalarGridSpec(
        num_scalar_prefetch=1,
        grid=(NT,),
        in_specs=[
            pl.BlockSpec((TM, H), lambda t, m: (t, 0)),
            pl.BlockSpec((TM, 128), lambda t, m: (t, 0)),
            pl.BlockSpec((1, 2 * I, H), lambda t, m: (m[t], 0, 0)),
            pl.BlockSpec((1, H, I), lambda t, m: (m[t], 0, 0)),
        ],
        out_specs=pl.BlockSpec((TM, H), lambda t, m: (t, 0)),
    )
    ys = pl.pallas_call(
        functools.partial(_gemm_body, I),
        grid_spec=grid_spec,
        out_shape=jax.ShapeDtypeStruct((NP, H), jnp.float32),
        compiler_params=pltpu.CompilerParams(
            dimension_semantics=("arbitrary",),
        ),
    )(meta, xs, ss, w3_w1_weight, w2_weight)

    combine = functools.partial(
        pl.kernel,
        mesh=mesh,
        out_type=jax.ShapeDtypeStruct((T, H), jnp.float32),
        scratch_types=[
            pltpu.VMEM((1, T // 32), jnp.int32),
            pltpu.VMEM((1, T // 32), jnp.int32),
            pltpu.VMEM((T // 32, H), jnp.float32),
            pltpu.VMEM((T // 32, H), jnp.float32),
            pltpu.SemaphoreType.DMA,
            pltpu.SemaphoreType.DMA,
        ],
    )(functools.partial(_combine_body, T, H))
    out = combine(ys, pos0, pos1)
    return out


# speedup vs baseline: 1.3838x; 1.0259x over previous
import functools

import jax
import jax.numpy as jnp
from jax import lax
from jax.experimental import pallas as pl
from jax.experimental.pallas import tpu as pltpu
from jax.experimental.pallas import tpu_sc as plsc

L = 16
TM = 512
NT = 16
NP = NT * TM


def _route_body(E, logits_ref, pos0_ref, pos1_ref, s0x_ref, s1x_ref, meta_ref):
    lg = logits_ref[...].astype(jnp.float32)
    T = lg.shape[0]
    iota = lax.broadcasted_iota(jnp.int32, (T, E), 1)
    v1 = jnp.max(lg, axis=1, keepdims=True)
    e1 = jnp.min(jnp.where(lg == v1, iota, E), axis=1, keepdims=True)
    lm = jnp.where(iota == e1, -3e38, lg)
    v2 = jnp.max(lm, axis=1, keepdims=True)
    e2 = jnp.min(jnp.where(lm == v2, iota, E), axis=1, keepdims=True)
    q = jnp.exp(v2 - v1)
    s1 = 1.0 / (1.0 + q)
    s2 = 1.0 - s1

    oh1 = (iota == e1).astype(jnp.bfloat16)
    oh2 = (iota == e2).astype(jnp.bfloat16)
    oh = oh1 + oh2
    r_i = lax.broadcasted_iota(jnp.int32, (T, T), 0)
    c_i = lax.broadcasted_iota(jnp.int32, (T, T), 1)
    tri = (c_i < r_i).astype(jnp.bfloat16)
    ranks = lax.dot_general(tri, oh, (((1,), (0,)), ((), ())),
                            preferred_element_type=jnp.float32)
    cnt = jnp.sum(oh.astype(jnp.float32), axis=0, keepdims=True)
    ptiles = jnp.ceil(cnt / TM)
    ce = lax.broadcasted_iota(jnp.int32, (E, E), 0)
    re = lax.broadcasted_iota(jnp.int32, (E, E), 1)
    trie = (ce < re).astype(jnp.float32)
    tstart = lax.dot_general(ptiles, trie, (((1,), (0,)), ((), ())),
                             preferred_element_type=jnp.float32)
    po = tstart * TM
    slot = po + ranks
    pos0 = jnp.sum(jnp.where(iota == e1, slot, 0.0), axis=1, keepdims=True)
    pos1 = jnp.sum(jnp.where(iota == e2, slot, 0.0), axis=1, keepdims=True)
    pos0_ref[...] = jnp.broadcast_to(pos0.astype(jnp.int32), (T, 8))
    pos1_ref[...] = jnp.broadcast_to(pos1.astype(jnp.int32), (T, 8))
    s0x_ref[...] = jnp.broadcast_to(s1, (T, 128)).astype(jnp.float32)
    s1x_ref[...] = jnp.broadcast_to(s2, (T, 128)).astype(jnp.float32)

    tcum = tstart + ptiles
    ident = (ce == re).astype(jnp.float32)
    tcol = lax.dot_general(ident, tcum, (((1,), (1,)), ((), ())),
                           preferred_element_type=jnp.float32)
    ntiles = jnp.sum(ptiles).astype(jnp.int32)
    jrow = lax.broadcasted_iota(jnp.int32, (E, 32), 1)
    texp = jnp.sum((tcol.astype(jnp.int32) <= jrow).astype(jnp.int32),
                   axis=0, keepdims=True)
    j32 = lax.broadcasted_iota(jnp.int32, (1, 32), 1)
    lastexp = jnp.sum(jnp.where(j32 == ntiles - 1, texp, 0))
    texp = jnp.minimum(texp, lastexp)
    mrow = jnp.where(j32 == NT, ntiles, jnp.where(j32 < NT, texp, 0))
    cnt_t = jnp.zeros((1, 32), jnp.float32)
    tst_t = jnp.zeros((1, 32), jnp.float32)
    iota_e = lax.broadcasted_iota(jnp.int32, (1, E), 1)
    for e in range(E):
        ce_s = jnp.sum(jnp.where(iota_e == e, cnt, 0.0))
        ts_s = jnp.sum(jnp.where(iota_e == e, tstart, 0.0))
        sel = texp == e
        cnt_t = cnt_t + jnp.where(sel, ce_s, 0.0)
        tst_t = tst_t + jnp.where(sel, ts_s, 0.0)
    rows = jnp.clip(cnt_t - (j32.astype(jnp.float32) - tst_t) * TM, 0.0, TM)
    rows = jnp.where(j32 < ntiles, rows, 0.0).astype(jnp.int32)
    si = lax.broadcasted_iota(jnp.int32, (8, 32), 0)
    meta_ref[...] = jnp.where(si == 0, jnp.broadcast_to(mrow, (8, 32)),
                              jnp.broadcast_to(rows, (8, 32)))


def _scatter_body(T, H, pos0_hbm, pos1_hbm, s0_hbm, s1_hbm, x_hbm,
                  xs_hbm, ss_hbm, posb0, posb1, sbuf0, sbuf1, xbuf,
                  sem, sem2, sem3):
    cid = lax.axis_index("c")
    sid = lax.axis_index("s")
    wid = sid * 2 + cid
    tpw = T // 32
    t0 = wid * tpw
    cx = pltpu.async_copy(x_hbm.at[pl.ds(t0, tpw)], xbuf, sem)
    c0 = pltpu.async_copy(s0_hbm.at[pl.ds(t0, tpw)], sbuf0, sem2)
    c1 = pltpu.async_copy(s1_hbm.at[pl.ds(t0, tpw)], sbuf1, sem3)
    pltpu.sync_copy(pos0_hbm.at[pl.ds(t0, tpw)], posb0.at[0])
    pltpu.sync_copy(pos1_hbm.at[pl.ds(t0, tpw)], posb1.at[0])
    c0.wait()
    c1.wait()
    cx.wait()
    w0 = pltpu.async_copy(sbuf0, ss_hbm.at[posb0.at[0]], sem2)
    w1 = pltpu.async_copy(sbuf1, ss_hbm.at[posb1.at[0]], sem3)
    ca = pltpu.async_copy(xbuf, xs_hbm.at[posb0.at[0]], sem)
    cb = pltpu.async_copy(xbuf, xs_hbm.at[posb1.at[0]], sem)
    w0.wait()
    w1.wait()
    ca.wait()
    cb.wait()


def _gemm_body(I, meta_ref, xs_ref, ss_ref, w31_ref, w2_ref, ys_ref):
    t = pl.program_id(0)
    rows = meta_ref[32 + t]

    @pl.when(rows > 0)
    def _():
        xb = xs_ref[...].astype(jnp.bfloat16)
        w31b = w31_ref[0].astype(jnp.bfloat16)
        dn = (((1,), (1,)), ((), ()))
        h = lax.dot_general(xb, w31b, dn, preferred_element_type=jnp.float32)
        x3 = h[:, :I]
        x1 = h[:, I:]
        a = (x1 * jax.nn.sigmoid(x1)) * x3
        w2b = w2_ref[0].astype(jnp.bfloat16)
        y = lax.dot_general(a.astype(jnp.bfloat16), w2b, dn,
                            preferred_element_type=jnp.float32)
        ys_ref[...] = y * ss_ref[:, 0:1]


def _combine_body(T, H, ys_hbm, pos0_hbm, pos1_hbm, out_hbm,
                  p0v, p1v, bufA, bufB, semA, semB):
    cid = lax.axis_index("c")
    sid = lax.axis_index("s")
    wid = sid * 2 + cid
    tpw = T // 32
    base = wid * tpw
    pltpu.sync_copy(pos0_hbm.at[pl.ds(base, tpw)], p0v.at[0])
    pltpu.sync_copy(pos1_hbm.at[pl.ds(base, tpw)], p1v.at[0])
    ca = pltpu.async_copy(ys_hbm.at[p0v.at[0]], bufA, semA)
    cb = pltpu.async_copy(ys_hbm.at[p1v.at[0]], bufB, semB)
    ca.wait()
    cb.wait()
    for r in range(64):
        def inner(k4, _, r=r):
            for u in range(4):
                sl = pl.ds(k4 * 4 * L + u * L, L)
                bufA[r, sl] = bufA[r, sl] + bufB[r, sl]
            return 0

        lax.fori_loop(0, H // (4 * L), inner, 0)
    pltpu.sync_copy(bufA, out_hbm.at[pl.ds(base, tpw)])


def kernel(x, router_logits, w3_w1_weight, w2_weight):
    T, H = x.shape
    E = w3_w1_weight.shape[0]
    I = w3_w1_weight.shape[1] // 2

    pos0f, pos1f, s0x, s1x, meta8 = pl.pallas_call(
        functools.partial(_route_body, E),
        out_shape=[
            jax.ShapeDtypeStruct((T, 8), jnp.int32),
            jax.ShapeDtypeStruct((T, 8), jnp.int32),
            jax.ShapeDtypeStruct((T, 128), jnp.float32),
            jax.ShapeDtypeStruct((T, 128), jnp.float32),
            jax.ShapeDtypeStruct((8, 32), jnp.int32),
        ],
    )(router_logits)
    pos0 = pos0f[:, 0]
    pos1 = pos1f[:, 0]
    s0c = s0x
    s1c = s1x
    meta = meta8[:2].reshape(64)

    mesh = plsc.VectorSubcoreMesh(core_axis_name="c", subcore_axis_name="s")

    scatter = functools.partial(
        pl.kernel,
        mesh=mesh,
        out_type=[
            jax.ShapeDtypeStruct((NP, H), jnp.float32),
            jax.ShapeDtypeStruct((NP, 128), jnp.float32),
        ],
        scratch_types=[
            pltpu.VMEM((1, T // 32), jnp.int32),
            pltpu.VMEM((1, T // 32), jnp.int32),
            pltpu.VMEM((T // 32, 128), jnp.float32),
            pltpu.VMEM((T // 32, 128), jnp.float32),
            pltpu.VMEM((T // 32, H), jnp.float32),
            pltpu.SemaphoreType.DMA,
            pltpu.SemaphoreType.DMA,
            pltpu.SemaphoreType.DMA,
        ],
    )(functools.partial(_scatter_body, T, H))
    xs, ss = scatter(pos0, pos1, s0c, s1c, x)

    grid_spec = pltpu.PrefetchScalarGridSpec(
        num_scalar_prefetch=1,
        grid=(NT,),
        in_specs=[
            pl.BlockSpec((TM, H), lambda t, m: (t, 0)),
            pl.BlockSpec((TM, 128), lambda t, m: (t, 0)),
            pl.BlockSpec((1, 2 * I, H), lambda t, m: (m[t], 0, 0)),
            pl.BlockSpec((1, H, I), lambda t, m: (m[t], 0, 0)),
        ],
        out_specs=pl.BlockSpec((TM, H), lambda t, m: (t, 0)),
    )
    ys = pl.pallas_call(
        functools.partial(_gemm_body, I),
        grid_spec=grid_spec,
        out_shape=jax.ShapeDtypeStruct((NP, H), jnp.float32),
        compiler_params=pltpu.CompilerParams(
            dimension_semantics=("arbitrary",),
        ),
    )(meta, xs, ss, w3_w1_weight, w2_weight)

    combine = functools.partial(
        pl.kernel,
        mesh=mesh,
        out_type=jax.ShapeDtypeStruct((T, H), jnp.float32),
        scratch_types=[
            pltpu.VMEM((1, T // 32), jnp.int32),
            pltpu.VMEM((1, T // 32), jnp.int32),
            pltpu.VMEM((T // 32, H), jnp.float32),
            pltpu.VMEM((T // 32, H), jnp.float32),
            pltpu.SemaphoreType.DMA,
            pltpu.SemaphoreType.DMA,
        ],
    )(functools.partial(_combine_body, T, H))
    out = combine(ys, pos0, pos1)
    return out
